# Initial kernel scaffold; baseline (speedup 1.0000x reference)
import jax
import jax.numpy as jnp
from jax.experimental import pallas as pl


def kernel(x, pos, batch, W, b):
    raise NotImplementedError("write your pallas kernel here")


# speedup vs baseline: 1.0560x; 1.0560x over previous
import jax
import jax.numpy as jnp
from jax.experimental import pallas as pl

_RATIO = 0.25
_K = 32
_CHUNK = 1250


def _mm_body(xp_ref, w_ref, g_ref):
    g_ref[...] = jnp.dot(xp_ref[...], w_ref[...],
                         preferred_element_type=jnp.float32)


def _g_matmul(xp, W):
    n, d = xp.shape
    dout = W.shape[1]
    rows = 2000
    return pl.pallas_call(
        _mm_body,
        grid=(n // rows,),
        in_specs=[pl.BlockSpec((rows, d), lambda i: (i, 0)),
                  pl.BlockSpec((d, dout), lambda i: (0, 0))],
        out_specs=pl.BlockSpec((rows, dout), lambda i: (i, 0)),
        out_shape=jax.ShapeDtypeStruct((n, dout), jnp.float32),
    )(xp, W)


def _knn_xla(pos, batch, pos_q, batch_q, k):
    nq = pos_q.shape[0]
    qs = pos_q.reshape(nq // _CHUNK, _CHUNK, 3)
    bqs = batch_q.reshape(nq // _CHUNK, _CHUNK)

    def body(args):
        q, bq = args
        d2 = jnp.sum((q[:, None, :] - pos[None, :, :]) ** 2, axis=-1)
        d2 = jnp.where(bq[:, None] != batch[None, :], jnp.inf, d2)
        _, idx = jax.lax.top_k(-d2, k)
        return idx

    return jax.lax.map(body, (qs, bqs)).reshape(nq, k)


def kernel(x, pos, batch, W, b):
    n, d = x.shape
    num_idxs = int(n * _RATIO)
    perm = jax.random.permutation(jax.random.key(42), n)[:num_idxs]
    idx = jnp.sort(perm)
    pos_q = jnp.take(pos, idx, axis=0)
    batch_q = jnp.take(batch, idx, axis=0)

    nn_idx = _knn_xla(pos, batch, pos_q, batch_q, _K)

    g = _g_matmul(jnp.concatenate([x, pos], axis=1), W)
    gmax = jnp.max(jnp.take(g, nn_idx, axis=0), axis=1)
    c = pos_q @ W[d:]
    out = jax.nn.relu(gmax - c + b)
    return (out, pos_q, batch_q)
